# baseline (speedup 1.0000x reference)
import numpy as np

import jax
import jax.numpy as jnp
from jax import lax
from jax.experimental import pallas as pl
from jax.experimental.pallas import tpu as pltpu
from jax.experimental.pallas import tpu_sc as plsc

TEMPERATURE = 2.0 / 3.0
K = 2048
ROWS = 64
COLS = 32768
L = 16
NV = COLS // L
NC = 2
NS = 16
HB = 4096
UNROLL = 16
MIN32 = np.int32(-(2 ** 31))


def _scan_desc(ref, nchunks, kk, acc0):
    lane = lax.broadcasted_iota(jnp.int32, (L,), 0)

    def body(j, carry):
        found, psel, above, acc = carry
        c = nchunks - 1 - j
        h = ref[pl.ds(c * L, L)]
        rev = lax.rev(h, (0,))
        cs = plsc.cumsum(rev)
        cum = cs + acc
        crossed = cum >= kk
        ncross = jnp.sum(crossed.astype(jnp.int32))
        any_crossed = ncross > 0
        t = L - ncross
        sel = lane == t
        above_here = jnp.sum(jnp.where(sel, cum - rev, 0))
        p_here = c * L + (L - 1 - t)
        is_here = jnp.logical_and(jnp.logical_not(found), any_crossed)
        psel = jnp.where(is_here, p_here, psel)
        above = jnp.where(is_here, above_here, above)
        found = jnp.logical_or(found, any_crossed)
        acc = acc + cs[L - 1]
        return found, psel, above, acc

    init = (jnp.bool_(False), jnp.int32(0), jnp.int32(0), acc0)
    _, psel, above, _ = lax.fori_loop(0, nchunks, body, init)
    return psel, above


def _find_bucket(hist, tot, nchunks, kk):
    lane = lax.broadcasted_iota(jnp.int32, (L,), 0)
    lane0 = lane == 0

    @plsc.parallel_loop(0, nchunks, unroll=UNROLL)
    def _(c):
        s = jnp.sum(hist[pl.ds(c * L, L)])
        plsc.store_compressed(tot.at[pl.ds(c, L)],
                              jnp.full((L,), s, jnp.int32), mask=lane0)

    cc, above_c = _scan_desc(tot, nchunks // L, kk, jnp.int32(0))
    h = hist[pl.ds(cc * L, L)]
    rev = lax.rev(h, (0,))
    cs = plsc.cumsum(rev)
    cum = cs + above_c
    crossed = cum >= kk
    ncross = jnp.sum(crossed.astype(jnp.int32))
    t = L - ncross
    sel = lane == t
    above = jnp.sum(jnp.where(sel, cum - rev, 0))
    b = cc * L + (L - 1 - t)
    return b, above


def _zero(ref, n):
    zero = jnp.zeros((L,), jnp.int32)

    @plsc.parallel_loop(0, n, step=L, unroll=UNROLL)
    def _(i):
        ref[pl.ds(i, L)] = zero


def _key_of(v):
    bits = lax.bitcast_convert_type(v, jnp.int32)
    m = lax.shift_right_arithmetic(bits, 31)
    return bits ^ jnp.bitwise_or(m, MIN32)


def _sc_body(x_hbm, mask_hbm, s_hbm, xa, xc, sb, hist, tot,
             sem_x, sem_s, sem_m0, sem_m1):
    wid = lax.axis_index("s") * NC + lax.axis_index("c")
    ones_i = jnp.ones((L,), jnp.int32)
    zero_i = jnp.zeros((L,), jnp.int32)
    one_f = jnp.ones((L,), jnp.float32)
    zero_f = jnp.zeros((L,), jnp.float32)
    tempv = jnp.full((L,), TEMPERATURE, jnp.float32)

    def process_row(xb, r, sem_m):
        _zero(hist, HB)

        @plsc.parallel_loop(0, COLS, step=L, unroll=UNROLL)
        def _(i):
            v = xb[pl.ds(i, L)]
            key_u = _key_of(v)
            y = (v + 1.0) / tempv
            sb[pl.ds(i, L)] = 1.0 / (1.0 + jnp.exp(-y))
            b = lax.shift_right_logical(key_u, 20)
            plsc.addupdate_scatter(hist, [b], ones_i)

        s_dma = pltpu.make_async_copy(sb, s_hbm.at[r], sem_s)
        s_dma.start()

        b1, above1 = _find_bucket(hist, tot, HB // L, jnp.int32(K))
        kk2 = jnp.int32(K) - above1

        _zero(hist, HB)
        b1v = jnp.full((L,), b1, jnp.int32)

        @plsc.parallel_loop(0, COLS, step=L, unroll=UNROLL)
        def _(i):
            ku = _key_of(xb[pl.ds(i, L)])
            top = lax.shift_right_logical(ku, 20)
            mid = jnp.bitwise_and(lax.shift_right_logical(ku, 8),
                                  jnp.int32(0xFFF))
            plsc.addupdate_scatter(hist, [mid], ones_i, mask=top == b1v)

        b2, above2 = _find_bucket(hist, tot, HB // L, kk2)
        kk3 = kk2 - above2

        _zero(hist, 256)
        pref = jnp.bitwise_or(lax.shift_left(b1, 12), b2)
        prefv = jnp.full((L,), pref, jnp.int32)

        @plsc.parallel_loop(0, COLS, step=L, unroll=UNROLL)
        def _(i):
            ku = _key_of(xb[pl.ds(i, L)])
            hi = lax.shift_right_logical(ku, 8)
            low = jnp.bitwise_and(ku, jnp.int32(0xFF))
            plsc.addupdate_scatter(hist, [low], ones_i, mask=hi == prefv)

        b3, _ = _scan_desc(hist, 256 // L, kk3, jnp.int32(0))

        t_u = jnp.bitwise_or(lax.shift_left(b1, 20),
                             jnp.bitwise_or(lax.shift_left(b2, 8), b3))
        t_i = t_u ^ MIN32
        tvi = jnp.full((L,), t_i, jnp.int32)
        bits_t = jnp.where(tvi < 0, tvi ^ jnp.int32(0x7FFFFFFF), tvi)
        xt = lax.bitcast_convert_type(bits_t, jnp.float32)
        zt = 1.0 / (1.0 + jnp.exp(-((xt + 1.0) / tempv)))

        @plsc.parallel_loop(0, COLS, step=L, unroll=UNROLL,
                            carry=jnp.int32(0))
        def n_ge(i, acc):
            zv = sb[pl.ds(i, L)]
            ge = zv >= zt
            xb[pl.ds(i, L)] = jnp.where(ge, one_f, zero_f)
            return acc + plsc.all_reduce_population_count(ge)[0]

        @pl.when(n_ge != jnp.int32(K))
        def _():
            @plsc.parallel_loop(0, COLS, step=L, unroll=UNROLL,
                                carry=jnp.int32(0))
            def c_gt(i, acc):
                gt = sb[pl.ds(i, L)] > zt
                return acc + plsc.all_reduce_population_count(gt)[0]

            def body(i, j_rem):
                zv = sb[pl.ds(i * L, L)]
                m1 = zv > zt
                tie = zv == zt
                cs = plsc.cumsum(jnp.where(tie, ones_i, zero_i))
                sel = jnp.logical_and(tie, cs <= j_rem)
                xb[pl.ds(i * L, L)] = jnp.where(
                    jnp.logical_or(m1, sel), one_f, zero_f)
                ntie = cs[L - 1]
                return j_rem - jnp.minimum(ntie, j_rem)

            lax.fori_loop(0, NV, body, jnp.int32(K) - c_gt)

        mask_dma = pltpu.make_async_copy(xb, mask_hbm.at[r], sem_m)
        mask_dma.start()
        return s_dma, mask_dma

    r0 = wid * 2
    pltpu.sync_copy(x_hbm.at[r0], xa)
    x1_dma = pltpu.make_async_copy(x_hbm.at[r0 + 1], xc, sem_x)
    x1_dma.start()
    s0_dma, m0_dma = process_row(xa, r0, sem_m0)
    s0_dma.wait()
    x1_dma.wait()
    s1_dma, m1_dma = process_row(xc, r0 + 1, sem_m1)
    s1_dma.wait()
    m0_dma.wait()
    m1_dma.wait()


@jax.jit
def kernel(x):
    mesh = plsc.VectorSubcoreMesh(core_axis_name="c", subcore_axis_name="s")
    out = pl.kernel(
        _sc_body,
        out_type=(
            jax.ShapeDtypeStruct((ROWS, COLS), jnp.float32),
            jax.ShapeDtypeStruct((ROWS, COLS), jnp.float32),
        ),
        mesh=mesh,
        compiler_params=pltpu.CompilerParams(needs_layout_passes=False),
        scratch_types=[
            pltpu.VMEM((COLS,), jnp.float32),
            pltpu.VMEM((COLS,), jnp.float32),
            pltpu.VMEM((COLS,), jnp.float32),
            pltpu.VMEM((HB,), jnp.int32),
            pltpu.VMEM((272,), jnp.int32),
            pltpu.SemaphoreType.DMA,
            pltpu.SemaphoreType.DMA,
            pltpu.SemaphoreType.DMA,
            pltpu.SemaphoreType.DMA,
        ],
    )(x)
    return out


# speedup vs baseline: 1.0293x; 1.0293x over previous
import numpy as np

import jax
import jax.numpy as jnp
from jax import lax
from jax.experimental import pallas as pl
from jax.experimental.pallas import tpu as pltpu
from jax.experimental.pallas import tpu_sc as plsc

TEMPERATURE = 2.0 / 3.0
K = 2048
ROWS = 64
COLS = 32768
L = 16
NV = COLS // L
NC = 2
NS = 16
HB = 4096
UNROLL = 8
MIN32 = np.int32(-(2 ** 31))


def _scan_desc(ref, nchunks, kk, acc0):
    lane = lax.broadcasted_iota(jnp.int32, (L,), 0)

    def body(j, carry):
        found, psel, above, acc = carry
        c = nchunks - 1 - j
        h = ref[pl.ds(c * L, L)]
        rev = lax.rev(h, (0,))
        cs = plsc.cumsum(rev)
        cum = cs + acc
        crossed = cum >= kk
        ncross = jnp.sum(crossed.astype(jnp.int32))
        any_crossed = ncross > 0
        t = L - ncross
        sel = lane == t
        above_here = jnp.sum(jnp.where(sel, cum - rev, 0))
        p_here = c * L + (L - 1 - t)
        is_here = jnp.logical_and(jnp.logical_not(found), any_crossed)
        psel = jnp.where(is_here, p_here, psel)
        above = jnp.where(is_here, above_here, above)
        found = jnp.logical_or(found, any_crossed)
        acc = acc + cs[L - 1]
        return found, psel, above, acc

    init = (jnp.bool_(False), jnp.int32(0), jnp.int32(0), acc0)
    _, psel, above, _ = lax.fori_loop(0, nchunks, body, init)
    return psel, above


def _find_bucket(hist, tot, nchunks, kk):
    lane = lax.broadcasted_iota(jnp.int32, (L,), 0)
    lane0 = lane == 0

    @plsc.parallel_loop(0, nchunks, unroll=UNROLL)
    def _(c):
        s = jnp.sum(hist[pl.ds(c * L, L)])
        plsc.store_compressed(tot.at[pl.ds(c, L)],
                              jnp.full((L,), s, jnp.int32), mask=lane0)

    cc, above_c = _scan_desc(tot, nchunks // L, kk, jnp.int32(0))
    h = hist[pl.ds(cc * L, L)]
    rev = lax.rev(h, (0,))
    cs = plsc.cumsum(rev)
    cum = cs + above_c
    crossed = cum >= kk
    ncross = jnp.sum(crossed.astype(jnp.int32))
    t = L - ncross
    sel = lane == t
    above = jnp.sum(jnp.where(sel, cum - rev, 0))
    b = cc * L + (L - 1 - t)
    return b, above


def _zero(ref, n):
    zero = jnp.zeros((L,), jnp.int32)

    @plsc.parallel_loop(0, n, step=L, unroll=UNROLL)
    def _(i):
        ref[pl.ds(i, L)] = zero


def _key_of(v):
    bits = lax.bitcast_convert_type(v, jnp.int32)
    m = lax.shift_right_arithmetic(bits, 31)
    return bits ^ jnp.bitwise_or(m, MIN32)


def _sc_body(x_hbm, mask_hbm, s_hbm, xa, xc, sb, hist, tot,
             sem_x, sem_xa, sem_xb, sem_s, sem_m0, sem_m1):
    wid = lax.axis_index("s") * NC + lax.axis_index("c")
    ones_i = jnp.ones((L,), jnp.int32)
    zero_i = jnp.zeros((L,), jnp.int32)
    one_f = jnp.ones((L,), jnp.float32)
    zero_f = jnp.zeros((L,), jnp.float32)
    tempv = jnp.full((L,), TEMPERATURE, jnp.float32)

    def pass1(xb, lo, hi):
        @plsc.parallel_loop(lo, hi, step=L, unroll=UNROLL)
        def _(i):
            v = xb[pl.ds(i, L)]
            key_u = _key_of(v)
            y = (v + 1.0) / tempv
            sb[pl.ds(i, L)] = 1.0 / (1.0 + jnp.exp(-y))
            b = lax.shift_right_logical(key_u, 20)
            plsc.addupdate_scatter(hist, [b], ones_i)

    def process_row(xb, r, sem_m, x_wait=None):
        _zero(hist, HB)

        if x_wait is None:
            pass1(xb, 0, COLS)
        else:
            x_wait[0].wait()
            pass1(xb, 0, COLS // 2)
            x_wait[1].wait()
            pass1(xb, COLS // 2, COLS)

        s_dma = pltpu.make_async_copy(sb, s_hbm.at[r], sem_s)
        s_dma.start()

        b1, above1 = _find_bucket(hist, tot, HB // L, jnp.int32(K))
        kk2 = jnp.int32(K) - above1

        _zero(hist, HB)
        b1v = jnp.full((L,), b1, jnp.int32)

        @plsc.parallel_loop(0, COLS, step=L, unroll=UNROLL)
        def _(i):
            ku = _key_of(xb[pl.ds(i, L)])
            top = lax.shift_right_logical(ku, 20)
            mid = jnp.bitwise_and(lax.shift_right_logical(ku, 8),
                                  jnp.int32(0xFFF))
            plsc.addupdate_scatter(hist, [mid], ones_i, mask=top == b1v)

        b2, above2 = _find_bucket(hist, tot, HB // L, kk2)
        kk3 = kk2 - above2

        _zero(hist, 256)
        pref = jnp.bitwise_or(lax.shift_left(b1, 12), b2)
        prefv = jnp.full((L,), pref, jnp.int32)

        @plsc.parallel_loop(0, COLS, step=L, unroll=UNROLL)
        def _(i):
            ku = _key_of(xb[pl.ds(i, L)])
            hi = lax.shift_right_logical(ku, 8)
            low = jnp.bitwise_and(ku, jnp.int32(0xFF))
            plsc.addupdate_scatter(hist, [low], ones_i, mask=hi == prefv)

        b3, _ = _scan_desc(hist, 256 // L, kk3, jnp.int32(0))

        t_u = jnp.bitwise_or(lax.shift_left(b1, 20),
                             jnp.bitwise_or(lax.shift_left(b2, 8), b3))
        t_i = t_u ^ MIN32
        tvi = jnp.full((L,), t_i, jnp.int32)
        bits_t = jnp.where(tvi < 0, tvi ^ jnp.int32(0x7FFFFFFF), tvi)
        xt = lax.bitcast_convert_type(bits_t, jnp.float32)
        zt = 1.0 / (1.0 + jnp.exp(-((xt + 1.0) / tempv)))

        @plsc.parallel_loop(0, COLS, step=L, unroll=UNROLL,
                            carry=jnp.int32(0))
        def n_ge(i, acc):
            zv = sb[pl.ds(i, L)]
            ge = zv >= zt
            xb[pl.ds(i, L)] = jnp.where(ge, one_f, zero_f)
            return acc + plsc.all_reduce_population_count(ge)[0]

        @pl.when(n_ge != jnp.int32(K))
        def _():
            @plsc.parallel_loop(0, COLS, step=L, unroll=UNROLL,
                                carry=jnp.int32(0))
            def c_gt(i, acc):
                gt = sb[pl.ds(i, L)] > zt
                return acc + plsc.all_reduce_population_count(gt)[0]

            def body(i, j_rem):
                zv = sb[pl.ds(i * L, L)]
                m1 = zv > zt
                tie = zv == zt
                cs = plsc.cumsum(jnp.where(tie, ones_i, zero_i))
                sel = jnp.logical_and(tie, cs <= j_rem)
                xb[pl.ds(i * L, L)] = jnp.where(
                    jnp.logical_or(m1, sel), one_f, zero_f)
                ntie = cs[L - 1]
                return j_rem - jnp.minimum(ntie, j_rem)

            lax.fori_loop(0, NV, body, jnp.int32(K) - c_gt)

        mask_dma = pltpu.make_async_copy(xb, mask_hbm.at[r], sem_m)
        mask_dma.start()
        return s_dma, mask_dma

    r0 = wid * 2
    half = COLS // 2
    x0a = pltpu.make_async_copy(x_hbm.at[r0, pl.ds(0, half)],
                                xa.at[pl.ds(0, half)], sem_xa)
    x0b = pltpu.make_async_copy(x_hbm.at[r0, pl.ds(half, half)],
                                xa.at[pl.ds(half, half)], sem_xb)
    x0a.start()
    x0b.start()
    x1_dma = pltpu.make_async_copy(x_hbm.at[r0 + 1], xc, sem_x)
    x1_dma.start()
    s0_dma, m0_dma = process_row(xa, r0, sem_m0, x_wait=(x0a, x0b))
    s0_dma.wait()
    x1_dma.wait()
    s1_dma, m1_dma = process_row(xc, r0 + 1, sem_m1)
    s1_dma.wait()
    m0_dma.wait()
    m1_dma.wait()


@jax.jit
def kernel(x):
    mesh = plsc.VectorSubcoreMesh(core_axis_name="c", subcore_axis_name="s")
    out = pl.kernel(
        _sc_body,
        out_type=(
            jax.ShapeDtypeStruct((ROWS, COLS), jnp.float32),
            jax.ShapeDtypeStruct((ROWS, COLS), jnp.float32),
        ),
        mesh=mesh,
        compiler_params=pltpu.CompilerParams(needs_layout_passes=False),
        scratch_types=[
            pltpu.VMEM((COLS,), jnp.float32),
            pltpu.VMEM((COLS,), jnp.float32),
            pltpu.VMEM((COLS,), jnp.float32),
            pltpu.VMEM((HB,), jnp.int32),
            pltpu.VMEM((272,), jnp.int32),
            pltpu.SemaphoreType.DMA,
            pltpu.SemaphoreType.DMA,
            pltpu.SemaphoreType.DMA,
            pltpu.SemaphoreType.DMA,
            pltpu.SemaphoreType.DMA,
            pltpu.SemaphoreType.DMA,
        ],
    )(x)
    return out


# speedup vs baseline: 1.0512x; 1.0213x over previous
import numpy as np

import jax
import jax.numpy as jnp
from jax import lax
from jax.experimental import pallas as pl
from jax.experimental.pallas import tpu as pltpu
from jax.experimental.pallas import tpu_sc as plsc

TEMPERATURE = 2.0 / 3.0
K = 2048
ROWS = 64
COLS = 32768
L = 16
NV = COLS // L
NC = 2
NS = 16
HB = 4096
UNROLL = 8
MIN32 = np.int32(-(2 ** 31))


def _scan_desc(ref, nchunks, kk, acc0):
    lane = lax.broadcasted_iota(jnp.int32, (L,), 0)

    def body(j, carry):
        found, psel, above, acc = carry
        c = nchunks - 1 - j
        h = ref[pl.ds(c * L, L)]
        rev = lax.rev(h, (0,))
        cs = plsc.cumsum(rev)
        cum = cs + acc
        crossed = cum >= kk
        ncross = jnp.sum(crossed.astype(jnp.int32))
        any_crossed = ncross > 0
        t = L - ncross
        sel = lane == t
        above_here = jnp.sum(jnp.where(sel, cum - rev, 0))
        p_here = c * L + (L - 1 - t)
        is_here = jnp.logical_and(jnp.logical_not(found), any_crossed)
        psel = jnp.where(is_here, p_here, psel)
        above = jnp.where(is_here, above_here, above)
        found = jnp.logical_or(found, any_crossed)
        acc = acc + cs[L - 1]
        return found, psel, above, acc

    init = (jnp.bool_(False), jnp.int32(0), jnp.int32(0), acc0)
    _, psel, above, _ = lax.fori_loop(0, nchunks, body, init)
    return psel, above


def _find_bucket(hist, tot, nchunks, kk):
    lane = lax.broadcasted_iota(jnp.int32, (L,), 0)
    lane0 = lane == 0

    @plsc.parallel_loop(0, nchunks, unroll=UNROLL)
    def _(c):
        s = jnp.sum(hist[pl.ds(c * L, L)])
        plsc.store_compressed(tot.at[pl.ds(c, L)],
                              jnp.full((L,), s, jnp.int32), mask=lane0)

    cc, above_c = _scan_desc(tot, nchunks // L, kk, jnp.int32(0))
    h = hist[pl.ds(cc * L, L)]
    rev = lax.rev(h, (0,))
    cs = plsc.cumsum(rev)
    cum = cs + above_c
    crossed = cum >= kk
    ncross = jnp.sum(crossed.astype(jnp.int32))
    t = L - ncross
    sel = lane == t
    above = jnp.sum(jnp.where(sel, cum - rev, 0))
    b = cc * L + (L - 1 - t)
    return b, above


def _zero(ref, n):
    zero = jnp.zeros((L,), jnp.int32)

    @plsc.parallel_loop(0, n, step=L, unroll=UNROLL)
    def _(i):
        ref[pl.ds(i, L)] = zero


def _key_of(v):
    bits = lax.bitcast_convert_type(v, jnp.int32)
    m = lax.shift_right_arithmetic(bits, 31)
    return bits ^ jnp.bitwise_or(m, MIN32)


def _sc_body(x_hbm, mask_hbm, s_hbm, xa, xc, sb, hist, tot,
             sem_x, sem_s, sem_m0, sem_m1):
    wid = lax.axis_index("s") * NC + lax.axis_index("c")
    ones_i = jnp.ones((L,), jnp.int32)
    zero_i = jnp.zeros((L,), jnp.int32)
    one_f = jnp.ones((L,), jnp.float32)
    zero_f = jnp.zeros((L,), jnp.float32)
    tempv = jnp.full((L,), TEMPERATURE, jnp.float32)

    def process_row(xb, r, sem_m):
        _zero(hist, HB)

        @plsc.parallel_loop(0, COLS, step=L, unroll=UNROLL)
        def _(i):
            v = xb[pl.ds(i, L)]
            key_u = _key_of(v)
            y = (v + 1.0) / tempv
            sb[pl.ds(i, L)] = 1.0 / (1.0 + jnp.exp(-y))
            b = lax.shift_right_logical(key_u, 20)
            plsc.addupdate_scatter(hist, [b], ones_i)

        s_dma = pltpu.make_async_copy(sb, s_hbm.at[r], sem_s)
        s_dma.start()

        b1, above1 = _find_bucket(hist, tot, HB // L, jnp.int32(K))
        kk2 = jnp.int32(K) - above1

        _zero(hist, HB)
        b1v = jnp.full((L,), b1, jnp.int32)

        @plsc.parallel_loop(0, COLS, step=L, unroll=UNROLL)
        def _(i):
            ku = _key_of(xb[pl.ds(i, L)])
            top = lax.shift_right_logical(ku, 20)
            mid = jnp.bitwise_and(lax.shift_right_logical(ku, 8),
                                  jnp.int32(0xFFF))
            plsc.addupdate_scatter(hist, [mid], ones_i, mask=top == b1v)

        b2, above2 = _find_bucket(hist, tot, HB // L, kk2)
        kk3 = kk2 - above2

        _zero(hist, 256)
        pref = jnp.bitwise_or(lax.shift_left(b1, 12), b2)
        prefv = jnp.full((L,), pref, jnp.int32)

        @plsc.parallel_loop(0, COLS, step=L, unroll=UNROLL)
        def _(i):
            ku = _key_of(xb[pl.ds(i, L)])
            hi = lax.shift_right_logical(ku, 8)
            low = jnp.bitwise_and(ku, jnp.int32(0xFF))
            plsc.addupdate_scatter(hist, [low], ones_i, mask=hi == prefv)

        b3, _ = _scan_desc(hist, 256 // L, kk3, jnp.int32(0))

        t_u = jnp.bitwise_or(lax.shift_left(b1, 20),
                             jnp.bitwise_or(lax.shift_left(b2, 8), b3))
        t_i = t_u ^ MIN32
        tvi = jnp.full((L,), t_i, jnp.int32)
        bits_t = jnp.where(tvi < 0, tvi ^ jnp.int32(0x7FFFFFFF), tvi)
        xt = lax.bitcast_convert_type(bits_t, jnp.float32)
        zt = 1.0 / (1.0 + jnp.exp(-((xt + 1.0) / tempv)))

        @plsc.parallel_loop(0, COLS, step=L, unroll=UNROLL,
                            carry=jnp.int32(0))
        def n_ge(i, acc):
            zv = sb[pl.ds(i, L)]
            ge = zv >= zt
            xb[pl.ds(i, L)] = jnp.where(ge, one_f, zero_f)
            return acc + plsc.all_reduce_population_count(ge)[0]

        @pl.when(n_ge != jnp.int32(K))
        def _():
            @plsc.parallel_loop(0, COLS, step=L, unroll=UNROLL,
                                carry=jnp.int32(0))
            def c_gt(i, acc):
                gt = sb[pl.ds(i, L)] > zt
                return acc + plsc.all_reduce_population_count(gt)[0]

            def body(i, j_rem):
                zv = sb[pl.ds(i * L, L)]
                m1 = zv > zt
                tie = zv == zt
                cs = plsc.cumsum(jnp.where(tie, ones_i, zero_i))
                sel = jnp.logical_and(tie, cs <= j_rem)
                xb[pl.ds(i * L, L)] = jnp.where(
                    jnp.logical_or(m1, sel), one_f, zero_f)
                ntie = cs[L - 1]
                return j_rem - jnp.minimum(ntie, j_rem)

            lax.fori_loop(0, NV, body, jnp.int32(K) - c_gt)

        mask_dma = pltpu.make_async_copy(xb, mask_hbm.at[r], sem_m)
        mask_dma.start()
        return s_dma, mask_dma

    r0 = wid * 2
    pltpu.sync_copy(x_hbm.at[r0], xa)
    x1_dma = pltpu.make_async_copy(x_hbm.at[r0 + 1], xc, sem_x)
    x1_dma.start()
    s0_dma, m0_dma = process_row(xa, r0, sem_m0)
    s0_dma.wait()
    x1_dma.wait()
    s1_dma, m1_dma = process_row(xc, r0 + 1, sem_m1)
    s1_dma.wait()
    m0_dma.wait()
    m1_dma.wait()


@jax.jit
def kernel(x):
    mesh = plsc.VectorSubcoreMesh(core_axis_name="c", subcore_axis_name="s")
    out = pl.kernel(
        _sc_body,
        out_type=(
            jax.ShapeDtypeStruct((ROWS, COLS), jnp.float32),
            jax.ShapeDtypeStruct((ROWS, COLS), jnp.float32),
        ),
        mesh=mesh,
        compiler_params=pltpu.CompilerParams(needs_layout_passes=False),
        scratch_types=[
            pltpu.VMEM((COLS,), jnp.float32),
            pltpu.VMEM((COLS,), jnp.float32),
            pltpu.VMEM((COLS,), jnp.float32),
            pltpu.VMEM((HB,), jnp.int32),
            pltpu.VMEM((272,), jnp.int32),
            pltpu.SemaphoreType.DMA,
            pltpu.SemaphoreType.DMA,
            pltpu.SemaphoreType.DMA,
            pltpu.SemaphoreType.DMA,
        ],
    )(x)
    return out
